# baseline (speedup 1.0000x reference)
import functools

import jax
import jax.numpy as jnp
from jax import lax
from jax.experimental import pallas as pl
from jax.experimental.pallas import tpu as pltpu
from jax.experimental.pallas import tpu_sc as plsc

_LANES = 16


@functools.lru_cache(maxsize=None)
def _build(B, T, D, NC, NS):
    NW = NC * NS
    BW = B // NW
    RPS = 128
    BPS = RPS // T
    NSTEP = BW // BPS
    KD = D // _LANES
    CCH = 128
    NCCH = BW // CCH
    OT = T + 1
    mesh = plsc.VectorSubcoreMesh(core_axis_name="c", subcore_axis_name="s")

    @functools.partial(
        pl.kernel,
        out_type=jax.ShapeDtypeStruct((B * OT, D), jnp.float32),
        mesh=mesh,
        scratch_types=[
            pltpu.VMEM((NSTEP, RPS), jnp.int32),
            pltpu.VMEM((BW,), jnp.int32),
            pltpu.VMEM((8, D), jnp.float32),
            pltpu.VMEM((8, D), jnp.float32),
            pltpu.VMEM((T, D), jnp.float32),
            pltpu.VMEM((3, RPS, D), jnp.float32),
            pltpu.VMEM((CCH, D), jnp.float32),
            pltpu.VMEM((RPS,), jnp.int32),
            pltpu.VMEM((CCH,), jnp.int32),
            pltpu.VMEM((3, RPS), jnp.int32),
            pltpu.VMEM((1, CCH), jnp.int32),
            pltpu.SemaphoreType.DMA,
            pltpu.SemaphoreType.DMA,
            pltpu.SemaphoreType.DMA,
            pltpu.SemaphoreType.DMA,
            pltpu.SemaphoreType.DMA,
            pltpu.SemaphoreType.DMA,
            pltpu.SemaphoreType.DMA,
        ],
    )
    def emb_kernel(cat_hbm, tok_hbm, cat_tab, patch_tab, row_tab, col_tab,
                   out_hbm, tok_v, catv, row_v, col_v, pos_v, gbuf, catbuf,
                   tvec, cvec, idxv, oidx_c, g0, g1, g2, w0, w1, w2, ps):
        gsem = (g0, g1, g2)
        wsem = (w0, w1, w2)
        wid = lax.axis_index("s") * NC + lax.axis_index("c")
        base = wid * BW

        pltpu.sync_copy(tok_hbm.at[pl.ds(wid * NSTEP, NSTEP)], tok_v)
        pltpu.sync_copy(cat_hbm.at[pl.ds(base, BW)], catv)
        pltpu.sync_copy(row_tab, row_v)
        pltpu.sync_copy(col_tab, col_v)

        lane = lax.iota(jnp.int32, _LANES)
        for kk in range(RPS // _LANES):
            k = lane + kk * _LANES
            tvec[pl.ds(kk * _LANES, _LANES)] = (k >> 6) * OT + (k & (T - 1))
        for kk in range(CCH // _LANES):
            k = lane + kk * _LANES
            cvec[pl.ds(kk * _LANES, _LANES)] = k * OT

        def posbody(t, carry):
            r = t // 8
            c = t % 8
            for kk in range(KD):
                s = pl.ds(kk * _LANES, _LANES)
                pos_v[t, s] = row_v[r, s] + col_v[c, s]
            return carry

        lax.fori_loop(0, T, posbody, 0)

        def issue_gather(j, r):
            pltpu.async_copy(patch_tab.at[tok_v.at[j]], gbuf.at[r], gsem[r])

        def do_step(j, r):
            pltpu.make_async_copy(
                patch_tab.at[tok_v.at[j]], gbuf.at[r], gsem[r]).wait()

            def addbody(t, carry):
                tp = t & (T - 1)
                for kk in range(KD):
                    s = pl.ds(kk * _LANES, _LANES)
                    gbuf[r, t, s] = gbuf[r, t, s] + pos_v[tp, s]
                return carry

            lax.fori_loop(0, RPS, addbody, 0)

            obase = (base + BPS * j) * OT + 1
            for kk in range(RPS // _LANES):
                s = pl.ds(kk * _LANES, _LANES)
                idxv[r, s] = tvec[s] + obase
            pltpu.async_copy(gbuf.at[r], out_hbm.at[idxv.at[r]], wsem[r])

            r2 = (r + 2) % 3

            @pl.when(j >= 1)
            def _wait_prev():
                pltpu.make_async_copy(
                    gbuf.at[r2], out_hbm.at[idxv.at[r2]], wsem[r2]).wait()

            @pl.when(j <= NSTEP - 3)
            def _prefetch():
                issue_gather(j + 2, r2)

        issue_gather(0, 0)
        issue_gather(1, 1)

        def iterbody(i, carry):
            for u in range(3):
                do_step(3 * i + u, u)
            return carry

        lax.fori_loop(0, (NSTEP - 1) // 3, iterbody, 0)
        do_step(jnp.int32(NSTEP - 1), (NSTEP - 1) % 3)

        rl = (NSTEP - 1) % 3
        pltpu.make_async_copy(
            gbuf.at[rl], out_hbm.at[idxv.at[rl]], wsem[rl]).wait()

        for c in range(NCCH):
            cbase = (base + c * CCH) * OT
            for kk in range(CCH // _LANES):
                s = pl.ds(kk * _LANES, _LANES)
                oidx_c[0, s] = cvec[s] + cbase
            pltpu.async_copy(
                cat_tab.at[catv.at[pl.ds(c * CCH, CCH)]], catbuf, ps).wait()
            pltpu.async_copy(catbuf, out_hbm.at[oidx_c.at[0]], ps).wait()

    return emb_kernel


def kernel(cat_idx, tokens, category_embed, patch_embed, row_embed, col_embed):
    B, T = tokens.shape
    D = patch_embed.shape[1]
    info = plsc.get_sparse_core_info()
    tok = tokens.astype(jnp.int32).reshape(B * T // 128, 128)
    f = _build(B, T, D, info.num_cores, info.num_subcores)
    out2d = f(cat_idx.astype(jnp.int32), tok, category_embed, patch_embed,
              row_embed, col_embed)
    return out2d.reshape(B, T + 1, D)


# speedup vs baseline: 2.6705x; 2.6705x over previous
import functools

import jax
import jax.numpy as jnp
from jax import lax
from jax.experimental import pallas as pl
from jax.experimental.pallas import tpu as pltpu
from jax.experimental.pallas import tpu_sc as plsc

_LANES = 16


@functools.lru_cache(maxsize=None)
def _build(B, T, D, NC, NS):
    NW = NC * NS
    BW = B // NW
    RPS = 128
    BPS = RPS // T
    NSTEP = BW // BPS
    KD = D // _LANES
    SPC = 32
    BPC = SPC * BPS
    NCH = NSTEP // SPC
    OT = T + 1
    mesh = plsc.VectorSubcoreMesh(core_axis_name="c", subcore_axis_name="s")

    @functools.partial(
        pl.kernel,
        out_type=jax.ShapeDtypeStruct((B, OT, D), jnp.float32),
        mesh=mesh,
        scratch_types=[
            pltpu.VMEM((3, SPC, RPS), jnp.int32),
            pltpu.VMEM((NCH, BPC), jnp.int32),
            pltpu.VMEM((3, BPC, D), jnp.float32),
            pltpu.VMEM((8, D), jnp.float32),
            pltpu.VMEM((8, D), jnp.float32),
            pltpu.VMEM((T, D), jnp.float32),
            pltpu.VMEM((2, RPS, D), jnp.float32),
            pltpu.VMEM((2, BPS, OT, D), jnp.float32),
            pltpu.SemaphoreType.DMA,
            pltpu.SemaphoreType.DMA,
            pltpu.SemaphoreType.DMA,
            pltpu.SemaphoreType.DMA,
            pltpu.SemaphoreType.DMA,
            pltpu.SemaphoreType.DMA,
        ],
    )
    def emb_kernel(cat_hbm, tok_hbm, cat_tab, patch_tab, row_tab, col_tab,
                   out_hbm, tokc, catv, catb, row_v, col_v, pos_v, gbuf,
                   obuf, ts, cs, g0, g1, o0, o1):
        gsem = (g0, g1)
        osem = (o0, o1)
        wid = lax.axis_index("s") * NC + lax.axis_index("c")
        base = wid * BW

        def issue_tok(c):
            pltpu.async_copy(
                tok_hbm.at[pl.ds(wid * NSTEP + c * SPC, SPC)],
                tokc.at[c % 3], ts)

        def wait_tok():
            pltpu.make_async_copy(
                tok_hbm.at[pl.ds(0, SPC)], tokc.at[0], ts).wait()

        def issue_cat(c):
            pltpu.async_copy(cat_tab.at[catv.at[c]], catb.at[c % 3], cs)

        def wait_cat():
            pltpu.make_async_copy(
                cat_tab.at[catv.at[0]], catb.at[0], cs).wait()

        def issue_gather(j, p):
            c3 = (j // SPC) % 3
            pltpu.async_copy(
                patch_tab.at[tokc.at[c3, j % SPC]], gbuf.at[p], gsem[p])

        def wait_gather(p):
            pltpu.make_async_copy(
                patch_tab.at[tokc.at[0, 0]], gbuf.at[p], gsem[p]).wait()

        def issue_write(j, p):
            pltpu.async_copy(obuf.at[p], out_hbm.at[pl.ds(base + BPS * j,
                                                          BPS)], osem[p])

        def wait_write(p):
            pltpu.make_async_copy(
                obuf.at[p], out_hbm.at[pl.ds(0, BPS)], osem[p]).wait()

        pltpu.sync_copy(cat_hbm.at[pl.ds(wid * NCH, NCH)], catv)
        issue_tok(0)
        issue_cat(0)
        pltpu.sync_copy(row_tab, row_v)
        pltpu.sync_copy(col_tab, col_v)

        def posbody(t, carry):
            r = t // 8
            c = t % 8
            for kk in range(KD):
                s = pl.ds(kk * _LANES, _LANES)
                pos_v[t, s] = row_v[r, s] + col_v[c, s]
            return carry

        lax.fori_loop(0, T, posbody, 0)

        wait_tok()
        wait_cat()
        issue_tok(1)
        issue_cat(1)
        wait_tok()
        wait_cat()
        issue_tok(2)
        issue_cat(2)
        issue_gather(jnp.int32(0), 0)
        issue_gather(jnp.int32(1), 1)

        def do_step(j, p):
            @pl.when((j > 0) & (j % SPC == 0))
            def _chunk():
                c = j // SPC

                @pl.when(c + 1 <= NCH - 1)
                def _w():
                    wait_tok()
                    wait_cat()

                @pl.when(c + 2 <= NCH - 1)
                def _i():
                    issue_tok(c + 2)
                    issue_cat(c + 2)

            wait_gather(p)

            @pl.when(j >= 2)
            def _wait_slab():
                wait_write(p)

            b0 = BPS * j
            cb = (b0 // BPC) % 3
            for h in range(BPS):
                r64 = (b0 + h) % BPC
                for kk in range(KD):
                    s = pl.ds(kk * _LANES, _LANES)
                    obuf[p, h, 0, s] = catb[cb, r64, s]

            def addbody(t, carry):
                for h in range(BPS):
                    for kk in range(KD):
                        s = pl.ds(kk * _LANES, _LANES)
                        obuf[p, h, t + 1, s] = (gbuf[p, h * T + t, s]
                                                + pos_v[t, s])
                return carry

            lax.fori_loop(0, T, addbody, 0)

            issue_write(j, p)

            @pl.when(j <= NSTEP - 3)
            def _prefetch():
                issue_gather(j + 2, p)

        def iterbody(i, carry):
            for u in range(2):
                do_step(2 * i + u, u)
            return carry

        lax.fori_loop(0, NSTEP // 2, iterbody, 0)
        wait_write(0)
        wait_write(1)

    return emb_kernel


def kernel(cat_idx, tokens, category_embed, patch_embed, row_embed, col_embed):
    B, T = tokens.shape
    D = patch_embed.shape[1]
    info = plsc.get_sparse_core_info()
    tok = tokens.astype(jnp.int32).reshape(B * T // 128, 128)
    cat2 = cat_idx.astype(jnp.int32).reshape(B // 64, 64)
    f = _build(B, T, D, info.num_cores, info.num_subcores)
    return f(cat2, tok, category_embed, patch_embed, row_embed, col_embed)


# speedup vs baseline: 5.2744x; 1.9750x over previous
import functools

import jax
import jax.numpy as jnp
from jax import lax
from jax.experimental import pallas as pl
from jax.experimental.pallas import tpu as pltpu
from jax.experimental.pallas import tpu_sc as plsc

_LANES = 16


@functools.lru_cache(maxsize=None)
def _build(B, T, D, NC, NS):
    NW = NC * NS
    BW = B // NW
    RPS = 128
    NCHK = BW // RPS
    OT = T + 1
    NSTEP = OT * NCHK
    KD = D // _LANES
    mesh = plsc.VectorSubcoreMesh(core_axis_name="c", subcore_axis_name="s")

    @functools.partial(
        pl.kernel,
        out_type=jax.ShapeDtypeStruct((OT * B, D), jnp.float32),
        mesh=mesh,
        scratch_types=[
            pltpu.VMEM((NCHK, RPS), jnp.int32),
            pltpu.VMEM((T * NCHK, RPS), jnp.int32),
            pltpu.VMEM((8, D), jnp.float32),
            pltpu.VMEM((8, D), jnp.float32),
            pltpu.VMEM((T, D), jnp.float32),
            pltpu.VMEM((NCHK, RPS, D), jnp.float32),
            pltpu.SemaphoreType.DMA,
            pltpu.SemaphoreType.DMA,
            pltpu.SemaphoreType.DMA,
            pltpu.SemaphoreType.DMA,
            pltpu.SemaphoreType.DMA,
            pltpu.SemaphoreType.DMA,
            pltpu.SemaphoreType.DMA,
            pltpu.SemaphoreType.DMA,
        ],
    )
    def emb_kernel(cidx_hbm, tidx_hbm, cat_tab, patch_tab, row_tab,
                   col_tab, out_hbm, cativ, tokiv, row_v, col_v, pos_v, gbuf,
                   g0, g1, g2, g3, w0, w1, w2, w3):
        gsem = (g0, g1, g2, g3)
        wsem = (w0, w1, w2, w3)
        wid = lax.axis_index("s") * NC + lax.axis_index("c")
        base = wid * BW

        pltpu.sync_copy(cidx_hbm.at[wid], cativ)
        pltpu.sync_copy(tidx_hbm.at[wid], tokiv)
        pltpu.sync_copy(row_tab, row_v)
        pltpu.sync_copy(col_tab, col_v)

        def posbody(t, carry):
            r = t // 8
            c = t % 8
            for kk in range(KD):
                s = pl.ds(kk * _LANES, _LANES)
                pos_v[t, s] = row_v[r, s] + col_v[c, s]
            return carry

        lax.fori_loop(0, T, posbody, 0)

        def issue_gather(j, p):
            @pl.when(j <= NCHK - 1)
            def _cat():
                pltpu.async_copy(
                    cat_tab.at[cativ.at[j]], gbuf.at[p], gsem[p])

            @pl.when(j >= NCHK)
            def _patch():
                pltpu.async_copy(
                    patch_tab.at[tokiv.at[j - NCHK]], gbuf.at[p], gsem[p])

        def wait_gather(p):
            pltpu.make_async_copy(
                patch_tab.at[tokiv.at[0]], gbuf.at[p], gsem[p]).wait()

        def wait_write(p):
            pltpu.make_async_copy(
                gbuf.at[p], out_hbm.at[pl.ds(0, RPS)], wsem[p]).wait()

        issue_gather(jnp.int32(0), 0)
        issue_gather(jnp.int32(1), 1)

        def iterbody(i, carry):
            for u in range(NCHK):
                j = NCHK * i + u
                wait_gather(u)

                @pl.when(i >= 1)
                def _add():
                    posk = tuple(pos_v[i - 1, pl.ds(kk * _LANES, _LANES)]
                                 for kk in range(KD))

                    def addbody(r, pk):
                        for kk in range(KD):
                            s = pl.ds(kk * _LANES, _LANES)
                            gbuf[u, r, s] = gbuf[u, r, s] + pk[kk]
                        return pk

                    lax.fori_loop(0, RPS, addbody, posk)

                pltpu.async_copy(
                    gbuf.at[u],
                    out_hbm.at[pl.ds(i * B + base + u * RPS, RPS)],
                    wsem[u])

                u2 = (u + 2) % NCHK

                @pl.when(j >= 2)
                def _drain():
                    wait_write(u2)

                @pl.when(j <= NSTEP - 3)
                def _prefetch():
                    issue_gather(j + 2, u2)
            return carry

        lax.fori_loop(0, OT, iterbody, 0)
        wait_write(2)
        wait_write(3)

    return emb_kernel


def kernel(cat_idx, tokens, category_embed, patch_embed, row_embed, col_embed):
    B, T = tokens.shape
    D = patch_embed.shape[1]
    info = plsc.get_sparse_core_info()
    NW = info.num_cores * info.num_subcores
    BW = B // NW
    NCHK = BW // 128
    catr = cat_idx.astype(jnp.int32).reshape(NW, NCHK, 128)
    tokr = (tokens.astype(jnp.int32)
            .reshape(NW, NCHK, 128, T)
            .transpose(0, 3, 1, 2)
            .reshape(NW, T * NCHK, 128))
    f = _build(B, T, D, info.num_cores, info.num_subcores)
    out2d = f(catr, tokr, category_embed, patch_embed, row_embed, col_embed)
    return out2d.reshape(T + 1, B, D).transpose(1, 0, 2)


# speedup vs baseline: 5.3681x; 1.0178x over previous
import functools

import jax
import jax.numpy as jnp
from jax import lax
from jax.experimental import pallas as pl
from jax.experimental.pallas import tpu as pltpu
from jax.experimental.pallas import tpu_sc as plsc

_LANES = 16


@functools.lru_cache(maxsize=None)
def _build(B, T, D, NC, NS):
    NW = NC * NS
    BW = B // NW
    RPS = 128
    NCHK = BW // RPS
    OT = T + 1
    NSTEP = OT * NCHK
    KD = D // _LANES
    mesh = plsc.VectorSubcoreMesh(core_axis_name="c", subcore_axis_name="s")

    @functools.partial(
        pl.kernel,
        out_type=jax.ShapeDtypeStruct((OT * B, D), jnp.float32),
        mesh=mesh,
        scratch_types=[
            pltpu.VMEM((NCHK, RPS), jnp.int32),
            pltpu.VMEM((T * NCHK, RPS), jnp.int32),
            pltpu.VMEM((8, D), jnp.float32),
            pltpu.VMEM((8, D), jnp.float32),
            pltpu.VMEM((T, D), jnp.float32),
            pltpu.VMEM((NCHK, RPS, D), jnp.float32),
            pltpu.SemaphoreType.DMA,
            pltpu.SemaphoreType.DMA,
            pltpu.SemaphoreType.DMA,
            pltpu.SemaphoreType.DMA,
            pltpu.SemaphoreType.DMA,
            pltpu.SemaphoreType.DMA,
            pltpu.SemaphoreType.DMA,
            pltpu.SemaphoreType.DMA,
        ],
    )
    def emb_kernel(cidx_hbm, tidx_hbm, cat_tab, patch_tab, row_tab,
                   col_tab, out_hbm, cativ, tokiv, row_v, col_v, pos_v, gbuf,
                   g0, g1, g2, g3, w0, w1, w2, w3):
        gsem = (g0, g1, g2, g3)
        wsem = (w0, w1, w2, w3)
        wid = lax.axis_index("s") * NC + lax.axis_index("c")
        base = wid * BW

        pltpu.sync_copy(cidx_hbm.at[wid], cativ)
        pltpu.sync_copy(tidx_hbm.at[wid], tokiv)
        pltpu.sync_copy(row_tab, row_v)
        pltpu.sync_copy(col_tab, col_v)

        def posbody(t, carry):
            r = t // 8
            c = t % 8
            for kk in range(KD):
                s = pl.ds(kk * _LANES, _LANES)
                pos_v[t, s] = row_v[r, s] + col_v[c, s]
            return carry

        lax.fori_loop(0, T, posbody, 0)

        def issue_gather(j, p):
            @pl.when(j <= NCHK - 1)
            def _cat():
                pltpu.async_copy(
                    cat_tab.at[cativ.at[j]], gbuf.at[p], gsem[p])

            @pl.when(j >= NCHK)
            def _patch():
                pltpu.async_copy(
                    patch_tab.at[tokiv.at[j - NCHK]], gbuf.at[p], gsem[p])

        def wait_gather(p):
            pltpu.make_async_copy(
                patch_tab.at[tokiv.at[0]], gbuf.at[p], gsem[p]).wait()

        def wait_write(p):
            pltpu.make_async_copy(
                gbuf.at[p], out_hbm.at[pl.ds(0, RPS)], wsem[p]).wait()

        issue_gather(jnp.int32(0), 0)
        issue_gather(jnp.int32(1), 1)

        def iterbody(i, carry):
            for u in range(NCHK):
                j = NCHK * i + u
                wait_gather(u)

                u2 = (u + 2) % NCHK

                @pl.when(j >= 2)
                def _drain():
                    wait_write(u2)

                @pl.when(j <= NSTEP - 3)
                def _prefetch():
                    issue_gather(j + 2, u2)

                @pl.when(i >= 1)
                def _add():
                    posk = tuple(pos_v[i - 1, pl.ds(kk * _LANES, _LANES)]
                                 for kk in range(KD))

                    def addbody(r, pk):
                        for kk in range(KD):
                            s = pl.ds(kk * _LANES, _LANES)
                            gbuf[u, r, s] = gbuf[u, r, s] + pk[kk]
                        return pk

                    lax.fori_loop(0, RPS, addbody, posk)

                pltpu.async_copy(
                    gbuf.at[u],
                    out_hbm.at[pl.ds(i * B + base + u * RPS, RPS)],
                    wsem[u])
            return carry

        lax.fori_loop(0, OT, iterbody, 0)
        wait_write(2)
        wait_write(3)

    return emb_kernel


def kernel(cat_idx, tokens, category_embed, patch_embed, row_embed, col_embed):
    B, T = tokens.shape
    D = patch_embed.shape[1]
    info = plsc.get_sparse_core_info()
    NW = info.num_cores * info.num_subcores
    BW = B // NW
    NCHK = BW // 128
    catr = cat_idx.astype(jnp.int32).reshape(NW, NCHK, 128)
    tokr = (tokens.astype(jnp.int32)
            .reshape(NW, NCHK, 128, T)
            .transpose(0, 3, 1, 2)
            .reshape(NW, T * NCHK, 128))
    f = _build(B, T, D, info.num_cores, info.num_subcores)
    out2d = f(catr, tokr, category_embed, patch_embed, row_embed, col_embed)
    return out2d.reshape(T + 1, B, D).transpose(1, 0, 2)
